# Initial kernel scaffold; baseline (speedup 1.0000x reference)
import jax
import jax.numpy as jnp
from jax.experimental import pallas as pl


def kernel(nuclei, params):
    raise NotImplementedError("write your pallas kernel here")


# speedup vs baseline: 3.4430x; 3.4430x over previous
import numpy as np
import jax
import jax.numpy as jnp
from jax.experimental import pallas as pl
from jax.experimental.pallas import tpu as pltpu

_HI = jax.lax.Precision.HIGHEST


def _dot(a, b):
    return jnp.dot(a, b, precision=_HI)

N = 512
BI = 32
RBF_DIM = 32
RBF_CUTOFF = 10.0
POS_CUTOFF = 5.0
N_RAD = 6
INV_DEG = 1.0 / (N - 1)
_CHG = np.tile(np.array([1, 6, 7, 8], dtype=np.int32), N // 4)

_FR3 = (np.arange(1, RBF_DIM + 1, dtype=np.float32) * np.pi / RBF_CUTOFF
        ).reshape(1, 1, RBF_DIM)
_FR2 = _FR3.reshape(1, RBF_DIM)
_RBF_SCALE = float(np.sqrt(2.0 / RBF_CUTOFF))
_MUS = np.linspace(0.0, POS_CUTOFF, N_RAD, dtype=np.float32).reshape(1, N_RAD)
_GAMMA = float((N_RAD / POS_CUTOFF) ** 2)


def _msg_layer(nuc_ref, nucT, fr3, E0, n_embed, Ws, Wr, We, b1, V, c1,
               s_ref, hsum_ref):
    H = Ws.shape[1]
    S = _dot(n_embed, Ws) + b1
    R = _dot(n_embed, Wr)
    s_ref[:, :H] = S

    h_diag = jnp.tanh(S + R + E0)

    def tile_body(t, carry):
        i0 = t * BI
        xs = nuc_ref[pl.ds(i0, BI), :]
        acc = jnp.zeros((BI, N), jnp.float32)
        for ci in range(3):
            diff = xs[:, ci:ci + 1] - nucT[ci:ci + 1, :]
            acc = acc + diff * diff
        dist = jnp.sqrt(acc)
        d3 = jnp.maximum(dist, 1e-9)[:, :, None]
        rbf3 = _RBF_SCALE * jnp.sin(d3 * fr3) / d3
        rbf2 = rbf3.reshape(BI * N, RBF_DIM)
        E2 = _dot(rbf2, We)
        E3 = E2.reshape(BI, N, H)
        St = s_ref[pl.ds(i0, BI), :H]
        h3 = jnp.tanh(E3 + St[:, None, :] + R[None, :, :])
        hsum_ref[pl.ds(i0, BI), :H] = jnp.sum(h3, axis=1)
        return carry

    jax.lax.fori_loop(0, N // BI, tile_body, 0)
    h_sum = hsum_ref[:, :H]

    return _dot((h_sum - h_diag) * INV_DEG, V) + c1


def _gnn_kernel(refs):
    (nuc_ref, nucT_ref, chg_ref, biasrows_ref, mus_ref, fr_ref,
     e00_ref, e01_ref,
     Ws0, Wr0, We0, b0, V0, c0, Un0, Um0, ub0, U20, ub20,
     Ws1, Wr1, We1, b1, V1, c1, Un1, Um1, ub1, U21, ub21,
     Na, Nb, Nc, nb, N2, Ga, Gb, Gc, gb, G2, gb2,
     node_out_ref, glob_out_ref, s_ref, hsum_ref) = refs

    nuc = nuc_ref[...]
    nucT = nucT_ref[...]
    mus = mus_ref[...]
    fr3 = fr_ref[...].reshape(1, 1, RBF_DIM)

    center = jnp.mean(nuc, axis=0, keepdims=True)
    pos = nuc - center
    rr = jnp.sqrt(jnp.sum(pos * pos, axis=1, keepdims=True))
    safe_r = jnp.maximum(rr, 1e-9)
    u = pos / safe_r
    rad = jnp.exp(-_GAMMA * (rr - mus) ** 2)
    env = 0.5 * (jnp.cos(np.pi * jnp.clip(rr / POS_CUTOFF, 0.0, 1.0)) + 1.0)
    rad = rad * env
    x, y, z = u[:, 0:1], u[:, 1:2], u[:, 2:3]
    sph = [0.28209479177387814 * jnp.ones_like(x),
           0.4886025119029199 * y,
           0.4886025119029199 * z,
           0.4886025119029199 * x,
           1.0925484305920792 * x * y,
           1.0925484305920792 * y * z,
           0.31539156525252005 * (3.0 * z ** 2 - 1.0)]
    pos_embed = jnp.concatenate([s * rad for s in sph], axis=1)

    n0 = jnp.concatenate([chg_ref[...], pos_embed], axis=1)

    msg0 = _msg_layer(nuc_ref, nucT, fr3, e00_ref[...], n0,
                      Ws0[...], Wr0[...], We0[...], b0[...], V0[...], c0[...],
                      s_ref, hsum_ref)
    uh0 = jnp.tanh(_dot(n0, Un0[...]) + _dot(msg0, Um0[...]) + ub0[...])
    n1 = _dot(uh0, U20[...]) + ub20[...]

    msg1 = _msg_layer(nuc_ref, nucT, fr3, e01_ref[...], n1,
                      Ws1[...], Wr1[...], We1[...], b1[...], V1[...], c1[...],
                      s_ref, hsum_ref)
    uh1 = jnp.tanh(_dot(n1, Un1[...]) + _dot(msg1, Um1[...]) + ub1[...])
    n2 = n1 + _dot(uh1, U21[...]) + ub21[...]

    nh = jnp.tanh(_dot(n0, Na[...]) + _dot(n1, Nb[...])
                  + _dot(n2, Nc[...]) + nb[...])
    node_out_ref[...] = _dot(nh, N2[...]) + biasrows_ref[...]

    m0 = jnp.mean(n0, axis=0, keepdims=True)
    m1 = jnp.mean(n1, axis=0, keepdims=True)
    m2 = jnp.mean(n2, axis=0, keepdims=True)
    gh = jnp.tanh(_dot(m0, Ga[...]) + _dot(m1, Gb[...])
                  + _dot(m2, Gc[...]) + gb[...])
    glob_out_ref[...] = _dot(gh, G2[...]) + gb2[...]


def _kernel_body(*refs):
    _gnn_kernel(refs)


def kernel(nuclei, params):
    nuclei = nuclei.reshape(-1, 3)
    p = params
    chg = jnp.asarray(_CHG)
    chg_embed = p['embed'][chg]
    bias_rows = p['node_bias0'][chg]

    W0, V0l = p['mp0'][0], p['mp0'][1]
    W1, V1l = p['mp1'][0], p['mp1'][1]
    U0a, U0b = p['up0'][0], p['up0'][1]
    U1a, U1b = p['up1'][0], p['up1'][1]
    Nla, Nlb = p['node_mlp0'][0], p['node_mlp0'][1]
    Gla, Glb = p['glob_mlp0'][0], p['glob_mlp0'][1]

    def row(b):
        return b.reshape(1, -1)

    d0 = np.float32(1e-9)
    rbf0 = jnp.asarray(_RBF_SCALE * np.sin(_FR2 * d0) / d0)
    e00 = rbf0 @ W0['W'][148:180]
    e01 = rbf0 @ W1['W'][128:160]

    args = [
        nuclei, nuclei.T, chg_embed, bias_rows,
        jnp.asarray(_MUS), jnp.asarray(_FR2), e00, e01,
        W0['W'][:74], W0['W'][74:148], W0['W'][148:180], row(W0['b']),
        V0l['W'], row(V0l['b']),
        U0a['W'][:74], U0a['W'][74:106], row(U0a['b']), U0b['W'], row(U0b['b']),
        W1['W'][:64], W1['W'][64:128], W1['W'][128:160], row(W1['b']),
        V1l['W'], row(V1l['b']),
        U1a['W'][:64], U1a['W'][64:96], row(U1a['b']), U1b['W'], row(U1b['b']),
        Nla['W'][:74], Nla['W'][74:138], Nla['W'][138:202], row(Nla['b']),
        Nlb['W'],
        Gla['W'][:74], Gla['W'][74:138], Gla['W'][138:202], row(Gla['b']),
        Glb['W'], row(Glb['b']),
    ]

    node_out, glob_out = pl.pallas_call(
        _kernel_body,
        out_shape=[jax.ShapeDtypeStruct((N, 64), jnp.float32),
                   jax.ShapeDtypeStruct((1, 1), jnp.float32)],
        scratch_shapes=[pltpu.VMEM((N, 128), jnp.float32),
                        pltpu.VMEM((N, 128), jnp.float32)],
    )(*args)
    return (node_out, glob_out.reshape(1))


# speedup vs baseline: 8.5748x; 2.4905x over previous
import numpy as np
import jax
import jax.numpy as jnp
from jax.experimental import pallas as pl
from jax.experimental.pallas import tpu as pltpu

_HI = jax.lax.Precision.HIGHEST


def _dot(a, b):
    return jnp.dot(a, b, precision=_HI)


_S_C1 = np.float32(6.28125)
_S_C2 = np.float32(0.0019353071)
_S_C3 = np.float32(2.0 * np.pi - 6.28125 - float(np.float32(0.0019353071)))
_S_INV2PI = np.float32(1.0 / (2.0 * np.pi))
_S_POLY = [np.float32(v) for v in (
    1.3609248309017567e-10, -2.4728719496274403e-08, 2.753594795310157e-06,
    -0.00019840544138572275, 0.008333321629423266, -0.16666665962394128,
    0.9999999993132955)]


def _sin_poly(x):
    n = jnp.round(x * _S_INV2PI)
    y = ((x - n * _S_C1) - n * _S_C2) - n * _S_C3
    z = y * y
    acc = jnp.full_like(z, _S_POLY[0])
    for c in _S_POLY[1:]:
        acc = acc * z + c
    return acc * y

N = 512
BI = 32
RBF_DIM = 32
RBF_CUTOFF = 10.0
POS_CUTOFF = 5.0
N_RAD = 6
INV_DEG = 1.0 / (N - 1)
_CHG = np.tile(np.array([1, 6, 7, 8], dtype=np.int32), N // 4)

_FR3 = (np.arange(1, RBF_DIM + 1, dtype=np.float32) * np.pi / RBF_CUTOFF
        ).reshape(1, 1, RBF_DIM)
_FR2 = _FR3.reshape(1, RBF_DIM)
_RBF_SCALE = float(np.sqrt(2.0 / RBF_CUTOFF))
_MUS = np.linspace(0.0, POS_CUTOFF, N_RAD, dtype=np.float32).reshape(1, N_RAD)
_GAMMA = float((N_RAD / POS_CUTOFF) ** 2)


def _msg_layer(nuc_ref, nucT, fr3, E0, n_embed, Ws, Wr, We, b1, V, c1,
               s_ref, hsum_ref):
    H = Ws.shape[1]
    S = _dot(n_embed, Ws) + b1
    R = _dot(n_embed, Wr)
    s_ref[:, :H] = S

    h_diag = jnp.tanh(S + R + E0)

    def tile_body(t, carry):
        i0 = t * BI
        xs = nuc_ref[pl.ds(i0, BI), :]
        acc = jnp.zeros((BI, N), jnp.float32)
        for ci in range(3):
            diff = xs[:, ci:ci + 1] - nucT[ci:ci + 1, :]
            acc = acc + diff * diff
        dist = jnp.sqrt(acc)
        d3 = jnp.maximum(dist, 1e-9)[:, :, None]
        rbf3 = _RBF_SCALE * _sin_poly(d3 * fr3) / d3
        rbf2 = rbf3.reshape(BI * N, RBF_DIM)
        E2 = _dot(rbf2, We)
        E3 = E2.reshape(BI, N, H)
        St = s_ref[pl.ds(i0, BI), :H]
        h3 = jnp.tanh(E3 + St[:, None, :] + R[None, :, :])
        hsum_ref[pl.ds(i0, BI), :H] = jnp.sum(h3, axis=1)
        return carry

    jax.lax.fori_loop(0, N // BI, tile_body, 0)
    h_sum = hsum_ref[:, :H]

    return _dot((h_sum - h_diag) * INV_DEG, V) + c1


def _gnn_kernel(refs):
    (nuc_ref, nucT_ref, chg_ref, biasrows_ref, mus_ref, fr_ref,
     e00_ref, e01_ref,
     Ws0, Wr0, We0, b0, V0, c0, Un0, Um0, ub0, U20, ub20,
     Ws1, Wr1, We1, b1, V1, c1, Un1, Um1, ub1, U21, ub21,
     Na, Nb, Nc, nb, N2, Ga, Gb, Gc, gb, G2, gb2,
     node_out_ref, glob_out_ref, s_ref, hsum_ref) = refs

    nuc = nuc_ref[...]
    nucT = nucT_ref[...]
    mus = mus_ref[...]
    fr3 = fr_ref[...].reshape(1, 1, RBF_DIM)

    center = jnp.mean(nuc, axis=0, keepdims=True)
    pos = nuc - center
    rr = jnp.sqrt(jnp.sum(pos * pos, axis=1, keepdims=True))
    safe_r = jnp.maximum(rr, 1e-9)
    u = pos / safe_r
    rad = jnp.exp(-_GAMMA * (rr - mus) ** 2)
    env = 0.5 * (jnp.cos(np.pi * jnp.clip(rr / POS_CUTOFF, 0.0, 1.0)) + 1.0)
    rad = rad * env
    x, y, z = u[:, 0:1], u[:, 1:2], u[:, 2:3]
    sph = [0.28209479177387814 * jnp.ones_like(x),
           0.4886025119029199 * y,
           0.4886025119029199 * z,
           0.4886025119029199 * x,
           1.0925484305920792 * x * y,
           1.0925484305920792 * y * z,
           0.31539156525252005 * (3.0 * z ** 2 - 1.0)]
    pos_embed = jnp.concatenate([s * rad for s in sph], axis=1)

    n0 = jnp.concatenate([chg_ref[...], pos_embed], axis=1)

    msg0 = _msg_layer(nuc_ref, nucT, fr3, e00_ref[...], n0,
                      Ws0[...], Wr0[...], We0[...], b0[...], V0[...], c0[...],
                      s_ref, hsum_ref)
    uh0 = jnp.tanh(_dot(n0, Un0[...]) + _dot(msg0, Um0[...]) + ub0[...])
    n1 = _dot(uh0, U20[...]) + ub20[...]

    msg1 = _msg_layer(nuc_ref, nucT, fr3, e01_ref[...], n1,
                      Ws1[...], Wr1[...], We1[...], b1[...], V1[...], c1[...],
                      s_ref, hsum_ref)
    uh1 = jnp.tanh(_dot(n1, Un1[...]) + _dot(msg1, Um1[...]) + ub1[...])
    n2 = n1 + _dot(uh1, U21[...]) + ub21[...]

    nh = jnp.tanh(_dot(n0, Na[...]) + _dot(n1, Nb[...])
                  + _dot(n2, Nc[...]) + nb[...])
    node_out_ref[...] = _dot(nh, N2[...]) + biasrows_ref[...]

    m0 = jnp.mean(n0, axis=0, keepdims=True)
    m1 = jnp.mean(n1, axis=0, keepdims=True)
    m2 = jnp.mean(n2, axis=0, keepdims=True)
    gh = jnp.tanh(_dot(m0, Ga[...]) + _dot(m1, Gb[...])
                  + _dot(m2, Gc[...]) + gb[...])
    glob_out_ref[...] = _dot(gh, G2[...]) + gb2[...]


def _kernel_body(*refs):
    _gnn_kernel(refs)


def kernel(nuclei, params):
    nuclei = nuclei.reshape(-1, 3)
    p = params
    chg = jnp.asarray(_CHG)
    chg_embed = p['embed'][chg]
    bias_rows = p['node_bias0'][chg]

    W0, V0l = p['mp0'][0], p['mp0'][1]
    W1, V1l = p['mp1'][0], p['mp1'][1]
    U0a, U0b = p['up0'][0], p['up0'][1]
    U1a, U1b = p['up1'][0], p['up1'][1]
    Nla, Nlb = p['node_mlp0'][0], p['node_mlp0'][1]
    Gla, Glb = p['glob_mlp0'][0], p['glob_mlp0'][1]

    def row(b):
        return b.reshape(1, -1)

    d0 = np.float32(1e-9)
    rbf0 = jnp.asarray(_RBF_SCALE * np.sin(_FR2 * d0) / d0)
    e00 = rbf0 @ W0['W'][148:180]
    e01 = rbf0 @ W1['W'][128:160]

    args = [
        nuclei, nuclei.T, chg_embed, bias_rows,
        jnp.asarray(_MUS), jnp.asarray(_FR2), e00, e01,
        W0['W'][:74], W0['W'][74:148], W0['W'][148:180], row(W0['b']),
        V0l['W'], row(V0l['b']),
        U0a['W'][:74], U0a['W'][74:106], row(U0a['b']), U0b['W'], row(U0b['b']),
        W1['W'][:64], W1['W'][64:128], W1['W'][128:160], row(W1['b']),
        V1l['W'], row(V1l['b']),
        U1a['W'][:64], U1a['W'][64:96], row(U1a['b']), U1b['W'], row(U1b['b']),
        Nla['W'][:74], Nla['W'][74:138], Nla['W'][138:202], row(Nla['b']),
        Nlb['W'],
        Gla['W'][:74], Gla['W'][74:138], Gla['W'][138:202], row(Gla['b']),
        Glb['W'], row(Glb['b']),
    ]

    node_out, glob_out = pl.pallas_call(
        _kernel_body,
        out_shape=[jax.ShapeDtypeStruct((N, 64), jnp.float32),
                   jax.ShapeDtypeStruct((1, 1), jnp.float32)],
        scratch_shapes=[pltpu.VMEM((N, 128), jnp.float32),
                        pltpu.VMEM((N, 128), jnp.float32)],
    )(*args)
    return (node_out, glob_out.reshape(1))


# speedup vs baseline: 12.3265x; 1.4375x over previous
import numpy as np
import jax
import jax.numpy as jnp
from jax.experimental import pallas as pl
from jax.experimental.pallas import tpu as pltpu

_HI = jax.lax.Precision.HIGHEST


def _dot(a, b):
    return jnp.dot(a, b, precision=_HI)


_S_C1 = np.float32(6.28125)
_S_C2 = np.float32(0.0019353071)
_S_INV2PI = np.float32(1.0 / (2.0 * np.pi))
_S_POLY = [np.float32(v) for v in (
    -2.0718568460132205e-08, 2.709474929678186e-06, -0.00019818289164461625,
    0.008332817527799885, -0.16666624551605752, 0.9999999448001017)]


def _sin_poly(x):
    n = jnp.round(x * _S_INV2PI)
    y = (x - n * _S_C1) - n * _S_C2
    z = y * y
    acc = jnp.full_like(z, _S_POLY[0])
    for c in _S_POLY[1:]:
        acc = acc * z + c
    return acc * y

N = 512
BI = 32
RBF_DIM = 32
RBF_CUTOFF = 10.0
POS_CUTOFF = 5.0
N_RAD = 6
INV_DEG = 1.0 / (N - 1)
_CHG = np.tile(np.array([1, 6, 7, 8], dtype=np.int32), N // 4)

_FR3 = (np.arange(1, RBF_DIM + 1, dtype=np.float32) * np.pi / RBF_CUTOFF
        ).reshape(1, 1, RBF_DIM)
_FR2 = _FR3.reshape(1, RBF_DIM)
_RBF_SCALE = float(np.sqrt(2.0 / RBF_CUTOFF))
_MUS = np.linspace(0.0, POS_CUTOFF, N_RAD, dtype=np.float32).reshape(1, N_RAD)
_GAMMA = float((N_RAD / POS_CUTOFF) ** 2)


def _msg_layer(nuc_ref, nucT, fr3, E0, n_embed, Ws, Wr, Weh, Wel, b1, V, c1,
               s_ref, hsum_ref):
    H = Ws.shape[1]
    S = _dot(n_embed, Ws) + b1
    R = _dot(n_embed, Wr)
    s_ref[:, :H] = S

    h_diag = jnp.tanh(S + R + E0)

    def tile_body(t, carry):
        i0 = t * BI
        xs = nuc_ref[pl.ds(i0, BI), :]
        acc = jnp.zeros((BI, N), jnp.float32)
        for ci in range(3):
            diff = xs[:, ci:ci + 1] - nucT[ci:ci + 1, :]
            acc = acc + diff * diff
        dist = jnp.sqrt(acc)
        d = jnp.maximum(dist, 1e-9)
        d3 = d[:, :, None]
        inv_d3 = (1.0 / d)[:, :, None]
        rbf3 = _sin_poly(d3 * fr3) * inv_d3
        rbf2 = rbf3.reshape(BI * N, RBF_DIM)
        rb_h = rbf2.astype(jnp.bfloat16)
        rb_l = (rbf2 - rb_h.astype(jnp.float32)).astype(jnp.bfloat16)
        f32 = jnp.float32
        E2 = (jnp.dot(rb_h, Weh, preferred_element_type=f32)
              + jnp.dot(rb_h, Wel, preferred_element_type=f32)
              + jnp.dot(rb_l, Weh, preferred_element_type=f32))
        E3 = E2.reshape(BI, N, H)
        St = s_ref[pl.ds(i0, BI), :H]
        h3 = jnp.tanh(E3 + St[:, None, :] + R[None, :, :])
        hsum_ref[pl.ds(i0, BI), :H] = jnp.sum(h3, axis=1)
        return carry

    jax.lax.fori_loop(0, N // BI, tile_body, 0)
    h_sum = hsum_ref[:, :H]

    return _dot((h_sum - h_diag) * INV_DEG, V) + c1


def _gnn_kernel(refs):
    (nuc_ref, nucT_ref, chg_ref, biasrows_ref, mus_ref, fr_ref,
     e00_ref, e01_ref,
     Ws0, Wr0, Weh0, Wel0, b0, V0, c0, Un0, Um0, ub0, U20, ub20,
     Ws1, Wr1, Weh1, Wel1, b1, V1, c1, Un1, Um1, ub1, U21, ub21,
     Na, Nb, Nc, nb, N2, Ga, Gb, Gc, gb, G2, gb2,
     node_out_ref, glob_out_ref, s_ref, hsum_ref) = refs

    nuc = nuc_ref[...]
    nucT = nucT_ref[...]
    mus = mus_ref[...]
    fr3 = fr_ref[...].reshape(1, 1, RBF_DIM)

    center = jnp.mean(nuc, axis=0, keepdims=True)
    pos = nuc - center
    rr = jnp.sqrt(jnp.sum(pos * pos, axis=1, keepdims=True))
    safe_r = jnp.maximum(rr, 1e-9)
    u = pos / safe_r
    rad = jnp.exp(-_GAMMA * (rr - mus) ** 2)
    env = 0.5 * (jnp.cos(np.pi * jnp.clip(rr / POS_CUTOFF, 0.0, 1.0)) + 1.0)
    rad = rad * env
    x, y, z = u[:, 0:1], u[:, 1:2], u[:, 2:3]
    sph = [0.28209479177387814 * jnp.ones_like(x),
           0.4886025119029199 * y,
           0.4886025119029199 * z,
           0.4886025119029199 * x,
           1.0925484305920792 * x * y,
           1.0925484305920792 * y * z,
           0.31539156525252005 * (3.0 * z ** 2 - 1.0)]
    pos_embed = jnp.concatenate([s * rad for s in sph], axis=1)

    n0 = jnp.concatenate([chg_ref[...], pos_embed], axis=1)

    msg0 = _msg_layer(nuc_ref, nucT, fr3, e00_ref[...], n0,
                      Ws0[...], Wr0[...], Weh0[...], Wel0[...], b0[...],
                      V0[...], c0[...], s_ref, hsum_ref)
    uh0 = jnp.tanh(_dot(n0, Un0[...]) + _dot(msg0, Um0[...]) + ub0[...])
    n1 = _dot(uh0, U20[...]) + ub20[...]

    msg1 = _msg_layer(nuc_ref, nucT, fr3, e01_ref[...], n1,
                      Ws1[...], Wr1[...], Weh1[...], Wel1[...], b1[...],
                      V1[...], c1[...], s_ref, hsum_ref)
    uh1 = jnp.tanh(_dot(n1, Un1[...]) + _dot(msg1, Um1[...]) + ub1[...])
    n2 = n1 + _dot(uh1, U21[...]) + ub21[...]

    nh = jnp.tanh(_dot(n0, Na[...]) + _dot(n1, Nb[...])
                  + _dot(n2, Nc[...]) + nb[...])
    node_out_ref[...] = _dot(nh, N2[...]) + biasrows_ref[...]

    m0 = jnp.mean(n0, axis=0, keepdims=True)
    m1 = jnp.mean(n1, axis=0, keepdims=True)
    m2 = jnp.mean(n2, axis=0, keepdims=True)
    gh = jnp.tanh(_dot(m0, Ga[...]) + _dot(m1, Gb[...])
                  + _dot(m2, Gc[...]) + gb[...])
    glob_out_ref[...] = _dot(gh, G2[...]) + gb2[...]


def _kernel_body(*refs):
    _gnn_kernel(refs)


def kernel(nuclei, params):
    nuclei = nuclei.reshape(-1, 3)
    p = params
    chg = jnp.asarray(_CHG)
    chg_embed = p['embed'][chg]
    bias_rows = p['node_bias0'][chg]

    W0, V0l = p['mp0'][0], p['mp0'][1]
    W1, V1l = p['mp1'][0], p['mp1'][1]
    U0a, U0b = p['up0'][0], p['up0'][1]
    U1a, U1b = p['up1'][0], p['up1'][1]
    Nla, Nlb = p['node_mlp0'][0], p['node_mlp0'][1]
    Gla, Glb = p['glob_mlp0'][0], p['glob_mlp0'][1]

    def row(b):
        return b.reshape(1, -1)

    d0 = np.float32(1e-9)
    rbf0 = jnp.asarray(_RBF_SCALE * np.sin(_FR2 * d0) / d0)
    e00 = rbf0 @ W0['W'][148:180]
    e01 = rbf0 @ W1['W'][128:160]

    def hi_lo(w):
        ws = w * _RBF_SCALE
        wh = ws.astype(jnp.bfloat16)
        return wh, (ws - wh.astype(jnp.float32)).astype(jnp.bfloat16)

    Weh0, Wel0 = hi_lo(W0['W'][148:180])
    Weh1, Wel1 = hi_lo(W1['W'][128:160])

    args = [
        nuclei, nuclei.T, chg_embed, bias_rows,
        jnp.asarray(_MUS), jnp.asarray(_FR2), e00, e01,
        W0['W'][:74], W0['W'][74:148], Weh0, Wel0, row(W0['b']),
        V0l['W'], row(V0l['b']),
        U0a['W'][:74], U0a['W'][74:106], row(U0a['b']), U0b['W'], row(U0b['b']),
        W1['W'][:64], W1['W'][64:128], Weh1, Wel1, row(W1['b']),
        V1l['W'], row(V1l['b']),
        U1a['W'][:64], U1a['W'][64:96], row(U1a['b']), U1b['W'], row(U1b['b']),
        Nla['W'][:74], Nla['W'][74:138], Nla['W'][138:202], row(Nla['b']),
        Nlb['W'],
        Gla['W'][:74], Gla['W'][74:138], Gla['W'][138:202], row(Gla['b']),
        Glb['W'], row(Glb['b']),
    ]

    node_out, glob_out = pl.pallas_call(
        _kernel_body,
        out_shape=[jax.ShapeDtypeStruct((N, 64), jnp.float32),
                   jax.ShapeDtypeStruct((1, 1), jnp.float32)],
        scratch_shapes=[pltpu.VMEM((N, 128), jnp.float32),
                        pltpu.VMEM((N, 128), jnp.float32)],
    )(*args)
    return (node_out, glob_out.reshape(1))


# speedup vs baseline: 22.3134x; 1.8102x over previous
import numpy as np
import jax
import jax.numpy as jnp
from jax.experimental import pallas as pl
from jax.experimental.pallas import tpu as pltpu

_HI = jax.lax.Precision.HIGHEST


def _dot(a, b):
    return jnp.dot(a, b, precision=_HI)


_S_C1 = np.float32(6.28125)
_S_C2 = np.float32(0.0019353071)
_S_INV2PI = np.float32(1.0 / (2.0 * np.pi))
_S_POLY = [np.float32(v) for v in (
    -2.0718568460132205e-08, 2.709474929678186e-06, -0.00019818289164461625,
    0.008332817527799885, -0.16666624551605752, 0.9999999448001017)]


def _sin_poly(x):
    n = jnp.round(x * _S_INV2PI)
    y = (x - n * _S_C1) - n * _S_C2
    z = y * y
    acc = jnp.full_like(z, _S_POLY[0])
    for c in _S_POLY[1:]:
        acc = acc * z + c
    return acc * y

N = 512
BI = 32
RBF_DIM = 32
RBF_CUTOFF = 10.0
POS_CUTOFF = 5.0
N_RAD = 6
INV_DEG = 1.0 / (N - 1)
_CHG = np.tile(np.array([1, 6, 7, 8], dtype=np.int32), N // 4)

_FR3 = (np.arange(1, RBF_DIM + 1, dtype=np.float32) * np.pi / RBF_CUTOFF
        ).reshape(1, 1, RBF_DIM)
_FR2 = _FR3.reshape(1, RBF_DIM)
_RBF_SCALE = float(np.sqrt(2.0 / RBF_CUTOFF))
_MUS = np.linspace(0.0, POS_CUTOFF, N_RAD, dtype=np.float32).reshape(1, N_RAD)
_GAMMA = float((N_RAD / POS_CUTOFF) ** 2)


def _msg_layer(nuc_ref, nucT, frcol, E0, n_embed, Ws, Wr, Weh, Wel, b1, V, c1,
               s_ref, hsum_ref):
    H = Ws.shape[1]
    S = _dot(n_embed, Ws) + b1
    R = _dot(n_embed, Wr)
    s_ref[:, :H] = S

    h_diag = jnp.tanh(S + R + E0)

    def tile_body(t, carry):
        i0 = t * BI
        xs = nuc_ref[pl.ds(i0, BI), :]
        acc = jnp.zeros((BI, N), jnp.float32)
        for ci in range(3):
            diff = xs[:, ci:ci + 1] - nucT[ci:ci + 1, :]
            acc = acc + diff * diff
        dist = jnp.sqrt(acc)
        d = jnp.maximum(dist, 1e-9)
        inv_d = 1.0 / d
        theta = d[:, None, :] * frcol[None, :, :]
        rbfP = _sin_poly(theta) * inv_d[:, None, :]
        rb_h = rbfP.astype(jnp.bfloat16)
        rb_l = (rbfP - rb_h.astype(jnp.float32)).astype(jnp.bfloat16)
        f32 = jnp.float32
        dn = (((1,), (0,)), ((), ()))
        E3 = (jax.lax.dot_general(rb_h, Weh, dn, preferred_element_type=f32)
              + jax.lax.dot_general(rb_h, Wel, dn, preferred_element_type=f32)
              + jax.lax.dot_general(rb_l, Weh, dn,
                                    preferred_element_type=f32))
        St = s_ref[pl.ds(i0, BI), :H]
        h3 = jnp.tanh(E3 + St[:, None, :] + R[None, :, :])
        hsum_ref[pl.ds(i0, BI), :H] = jnp.sum(h3, axis=1)
        return carry

    jax.lax.fori_loop(0, N // BI, tile_body, 0)
    h_sum = hsum_ref[:, :H]

    return _dot((h_sum - h_diag) * INV_DEG, V) + c1


def _gnn_kernel(refs):
    (nuc_ref, nucT_ref, chg_ref, biasrows_ref, mus_ref, fr_ref,
     e00_ref, e01_ref,
     Ws0, Wr0, Weh0, Wel0, b0, V0, c0, Un0, Um0, ub0, U20, ub20,
     Ws1, Wr1, Weh1, Wel1, b1, V1, c1, Un1, Um1, ub1, U21, ub21,
     Na, Nb, Nc, nb, N2, Ga, Gb, Gc, gb, G2, gb2,
     node_out_ref, glob_out_ref, s_ref, hsum_ref) = refs

    nuc = nuc_ref[...]
    nucT = nucT_ref[...]
    mus = mus_ref[...]
    frcol = fr_ref[...]

    center = jnp.mean(nuc, axis=0, keepdims=True)
    pos = nuc - center
    rr = jnp.sqrt(jnp.sum(pos * pos, axis=1, keepdims=True))
    safe_r = jnp.maximum(rr, 1e-9)
    u = pos / safe_r
    rad = jnp.exp(-_GAMMA * (rr - mus) ** 2)
    env = 0.5 * (jnp.cos(np.pi * jnp.clip(rr / POS_CUTOFF, 0.0, 1.0)) + 1.0)
    rad = rad * env
    x, y, z = u[:, 0:1], u[:, 1:2], u[:, 2:3]
    sph = [0.28209479177387814 * jnp.ones_like(x),
           0.4886025119029199 * y,
           0.4886025119029199 * z,
           0.4886025119029199 * x,
           1.0925484305920792 * x * y,
           1.0925484305920792 * y * z,
           0.31539156525252005 * (3.0 * z ** 2 - 1.0)]
    pos_embed = jnp.concatenate([s * rad for s in sph], axis=1)

    n0 = jnp.concatenate([chg_ref[...], pos_embed], axis=1)

    msg0 = _msg_layer(nuc_ref, nucT, frcol, e00_ref[...], n0,
                      Ws0[...], Wr0[...], Weh0[...], Wel0[...], b0[...],
                      V0[...], c0[...], s_ref, hsum_ref)
    uh0 = jnp.tanh(_dot(n0, Un0[...]) + _dot(msg0, Um0[...]) + ub0[...])
    n1 = _dot(uh0, U20[...]) + ub20[...]

    msg1 = _msg_layer(nuc_ref, nucT, frcol, e01_ref[...], n1,
                      Ws1[...], Wr1[...], Weh1[...], Wel1[...], b1[...],
                      V1[...], c1[...], s_ref, hsum_ref)
    uh1 = jnp.tanh(_dot(n1, Un1[...]) + _dot(msg1, Um1[...]) + ub1[...])
    n2 = n1 + _dot(uh1, U21[...]) + ub21[...]

    nh = jnp.tanh(_dot(n0, Na[...]) + _dot(n1, Nb[...])
                  + _dot(n2, Nc[...]) + nb[...])
    node_out_ref[...] = _dot(nh, N2[...]) + biasrows_ref[...]

    m0 = jnp.mean(n0, axis=0, keepdims=True)
    m1 = jnp.mean(n1, axis=0, keepdims=True)
    m2 = jnp.mean(n2, axis=0, keepdims=True)
    gh = jnp.tanh(_dot(m0, Ga[...]) + _dot(m1, Gb[...])
                  + _dot(m2, Gc[...]) + gb[...])
    glob_out_ref[...] = _dot(gh, G2[...]) + gb2[...]


def _kernel_body(*refs):
    _gnn_kernel(refs)


def kernel(nuclei, params):
    nuclei = nuclei.reshape(-1, 3)
    p = params
    chg = jnp.asarray(_CHG)
    chg_embed = p['embed'][chg]
    bias_rows = p['node_bias0'][chg]

    W0, V0l = p['mp0'][0], p['mp0'][1]
    W1, V1l = p['mp1'][0], p['mp1'][1]
    U0a, U0b = p['up0'][0], p['up0'][1]
    U1a, U1b = p['up1'][0], p['up1'][1]
    Nla, Nlb = p['node_mlp0'][0], p['node_mlp0'][1]
    Gla, Glb = p['glob_mlp0'][0], p['glob_mlp0'][1]

    def row(b):
        return b.reshape(1, -1)

    d0 = np.float32(1e-9)
    rbf0 = jnp.asarray(_RBF_SCALE * np.sin(_FR2 * d0) / d0)
    e00 = rbf0 @ W0['W'][148:180]
    e01 = rbf0 @ W1['W'][128:160]

    def hi_lo(w):
        ws = w * _RBF_SCALE
        wh = ws.astype(jnp.bfloat16)
        return wh, (ws - wh.astype(jnp.float32)).astype(jnp.bfloat16)

    Weh0, Wel0 = hi_lo(W0['W'][148:180])
    Weh1, Wel1 = hi_lo(W1['W'][128:160])

    args = [
        nuclei, nuclei.T, chg_embed, bias_rows,
        jnp.asarray(_MUS), jnp.asarray(_FR2.T.copy()), e00, e01,
        W0['W'][:74], W0['W'][74:148], Weh0, Wel0, row(W0['b']),
        V0l['W'], row(V0l['b']),
        U0a['W'][:74], U0a['W'][74:106], row(U0a['b']), U0b['W'], row(U0b['b']),
        W1['W'][:64], W1['W'][64:128], Weh1, Wel1, row(W1['b']),
        V1l['W'], row(V1l['b']),
        U1a['W'][:64], U1a['W'][64:96], row(U1a['b']), U1b['W'], row(U1b['b']),
        Nla['W'][:74], Nla['W'][74:138], Nla['W'][138:202], row(Nla['b']),
        Nlb['W'],
        Gla['W'][:74], Gla['W'][74:138], Gla['W'][138:202], row(Gla['b']),
        Glb['W'], row(Glb['b']),
    ]

    node_out, glob_out = pl.pallas_call(
        _kernel_body,
        out_shape=[jax.ShapeDtypeStruct((N, 64), jnp.float32),
                   jax.ShapeDtypeStruct((1, 1), jnp.float32)],
        scratch_shapes=[pltpu.VMEM((N, 128), jnp.float32),
                        pltpu.VMEM((N, 128), jnp.float32)],
    )(*args)
    return (node_out, glob_out.reshape(1))


# speedup vs baseline: 26.0417x; 1.1671x over previous
import numpy as np
import jax
import jax.numpy as jnp
from jax.experimental import pallas as pl
from jax.experimental.pallas import tpu as pltpu

_HI = jax.lax.Precision.HIGHEST


def _dot(a, b):
    return jnp.dot(a, b, precision=_HI)


_S_C1 = np.float32(6.28125)
_S_C2 = np.float32(0.0019353071)
_S_INV2PI = np.float32(1.0 / (2.0 * np.pi))
_S_POLY = [np.float32(v) for v in (
    -2.0718568460132205e-08, 2.709474929678186e-06, -0.00019818289164461625,
    0.008332817527799885, -0.16666624551605752, 0.9999999448001017)]


def _sin_poly(x):
    n = jnp.round(x * _S_INV2PI)
    y = (x - n * _S_C1) - n * _S_C2
    z = y * y
    acc = jnp.full_like(z, _S_POLY[0])
    for c in _S_POLY[1:]:
        acc = acc * z + c
    return acc * y

N = 512
BI = 32
RBF_DIM = 32
RBF_CUTOFF = 10.0
POS_CUTOFF = 5.0
N_RAD = 6
INV_DEG = 1.0 / (N - 1)
_CHG = np.tile(np.array([1, 6, 7, 8], dtype=np.int32), N // 4)

_FR3 = (np.arange(1, RBF_DIM + 1, dtype=np.float32) * np.pi / RBF_CUTOFF
        ).reshape(1, 1, RBF_DIM)
_FR2 = _FR3.reshape(1, RBF_DIM)
_RBF_SCALE = float(np.sqrt(2.0 / RBF_CUTOFF))
_MUS = np.linspace(0.0, POS_CUTOFF, N_RAD, dtype=np.float32).reshape(1, N_RAD)
_GAMMA = float((N_RAD / POS_CUTOFF) ** 2)


def _msg_layer(nuc_ref, nucT, frcol, E0, n_embed, Ws, Wr, Wef, b1,
               V, c1, s_ref, hsum_ref):
    H = Ws.shape[1]
    S = _dot(n_embed, Ws) + b1
    R = _dot(n_embed, Wr)
    s_ref[:, :H] = S

    h_diag = jnp.tanh(S + R + E0)

    def tile_body(t, carry):
        i0 = t * BI
        xs = nuc_ref[pl.ds(i0, BI), :]
        acc = jnp.zeros((BI, N), jnp.float32)
        for ci in range(3):
            diff = xs[:, ci:ci + 1] - nucT[ci:ci + 1, :]
            acc = acc + diff * diff
        dist = jnp.sqrt(acc)
        d = jnp.maximum(dist, 1e-9)
        inv_d = 1.0 / d
        theta = d[:, None, :] * frcol[None, :, :]
        rbfP = _sin_poly(theta) * inv_d[:, None, :]
        rb_h = rbfP.astype(jnp.bfloat16)
        rb_l = (rbfP - rb_h.astype(jnp.float32)).astype(jnp.bfloat16)
        lhs = jnp.concatenate([rb_h, rb_h, rb_l], axis=1)
        dn = (((1,), (0,)), ((), ()))
        E3 = jax.lax.dot_general(lhs, Wef, dn,
                                 preferred_element_type=jnp.float32)
        St = s_ref[pl.ds(i0, BI), :H]
        h3 = jnp.tanh(E3 + St[:, None, :] + R[None, :, :])
        hsum_ref[pl.ds(i0, BI), :H] = jnp.sum(h3, axis=1)
        return carry

    jax.lax.fori_loop(0, N // BI, tile_body, 0)
    h_sum = hsum_ref[:, :H]

    return _dot((h_sum - h_diag) * INV_DEG, V) + c1


def _gnn_kernel(refs):
    (nuc_ref, nucT_ref, chg_ref, biasrows_ref, mus_ref, fr_ref,
     e00_ref, e01_ref,
     Ws0, Wr0, Wef0, b0, V0, c0, Un0, Um0, ub0, U20, ub20,
     Ws1, Wr1, Wef1, b1, V1, c1, Un1, Um1, ub1, U21, ub21,
     Na, Nb, Nc, nb, N2, Ga, Gb, Gc, gb, G2, gb2,
     node_out_ref, glob_out_ref, s_ref, hsum_ref) = refs

    nuc = nuc_ref[...]
    nucT = nucT_ref[...]
    mus = mus_ref[...]
    frcol = fr_ref[...]

    center = jnp.mean(nuc, axis=0, keepdims=True)
    pos = nuc - center
    rr = jnp.sqrt(jnp.sum(pos * pos, axis=1, keepdims=True))
    safe_r = jnp.maximum(rr, 1e-9)
    u = pos / safe_r
    rad = jnp.exp(-_GAMMA * (rr - mus) ** 2)
    env = 0.5 * (jnp.cos(np.pi * jnp.clip(rr / POS_CUTOFF, 0.0, 1.0)) + 1.0)
    rad = rad * env
    x, y, z = u[:, 0:1], u[:, 1:2], u[:, 2:3]
    sph = [0.28209479177387814 * jnp.ones_like(x),
           0.4886025119029199 * y,
           0.4886025119029199 * z,
           0.4886025119029199 * x,
           1.0925484305920792 * x * y,
           1.0925484305920792 * y * z,
           0.31539156525252005 * (3.0 * z ** 2 - 1.0)]
    pos_embed = jnp.concatenate([s * rad for s in sph], axis=1)

    n0 = jnp.concatenate([chg_ref[...], pos_embed], axis=1)

    msg0 = _msg_layer(nuc_ref, nucT, frcol, e00_ref[...], n0,
                      Ws0[...], Wr0[...], Wef0[...], b0[...],
                      V0[...], c0[...], s_ref, hsum_ref)
    uh0 = jnp.tanh(_dot(n0, Un0[...]) + _dot(msg0, Um0[...]) + ub0[...])
    n1 = _dot(uh0, U20[...]) + ub20[...]

    msg1 = _msg_layer(nuc_ref, nucT, frcol, e01_ref[...], n1,
                      Ws1[...], Wr1[...], Wef1[...], b1[...],
                      V1[...], c1[...], s_ref, hsum_ref)
    uh1 = jnp.tanh(_dot(n1, Un1[...]) + _dot(msg1, Um1[...]) + ub1[...])
    n2 = n1 + _dot(uh1, U21[...]) + ub21[...]

    nh = jnp.tanh(_dot(n0, Na[...]) + _dot(n1, Nb[...])
                  + _dot(n2, Nc[...]) + nb[...])
    node_out_ref[...] = _dot(nh, N2[...]) + biasrows_ref[...]

    m0 = jnp.mean(n0, axis=0, keepdims=True)
    m1 = jnp.mean(n1, axis=0, keepdims=True)
    m2 = jnp.mean(n2, axis=0, keepdims=True)
    gh = jnp.tanh(_dot(m0, Ga[...]) + _dot(m1, Gb[...])
                  + _dot(m2, Gc[...]) + gb[...])
    glob_out_ref[...] = _dot(gh, G2[...]) + gb2[...]


def _kernel_body(*refs):
    _gnn_kernel(refs)


def kernel(nuclei, params):
    nuclei = nuclei.reshape(-1, 3)
    p = params
    chg = jnp.asarray(_CHG)
    chg_embed = p['embed'][chg]
    bias_rows = p['node_bias0'][chg]

    W0, V0l = p['mp0'][0], p['mp0'][1]
    W1, V1l = p['mp1'][0], p['mp1'][1]
    U0a, U0b = p['up0'][0], p['up0'][1]
    U1a, U1b = p['up1'][0], p['up1'][1]
    Nla, Nlb = p['node_mlp0'][0], p['node_mlp0'][1]
    Gla, Glb = p['glob_mlp0'][0], p['glob_mlp0'][1]

    def row(b):
        return b.reshape(1, -1)

    d0 = np.float32(1e-9)
    rbf0 = jnp.asarray(_RBF_SCALE * np.sin(_FR2 * d0) / d0)
    e00 = rbf0 @ W0['W'][148:180]
    e01 = rbf0 @ W1['W'][128:160]

    def stack_hi_lo(w):
        ws = w * _RBF_SCALE
        wh = ws.astype(jnp.bfloat16)
        wl = (ws - wh.astype(jnp.float32)).astype(jnp.bfloat16)
        return jnp.concatenate([wh, wl, wh], axis=0)

    Wef0 = stack_hi_lo(W0['W'][148:180])
    Wef1 = stack_hi_lo(W1['W'][128:160])

    args = [
        nuclei, nuclei.T, chg_embed, bias_rows,
        jnp.asarray(_MUS), jnp.asarray(_FR2.T.copy()), e00, e01,
        W0['W'][:74], W0['W'][74:148], Wef0, row(W0['b']),
        V0l['W'], row(V0l['b']),
        U0a['W'][:74], U0a['W'][74:106], row(U0a['b']), U0b['W'], row(U0b['b']),
        W1['W'][:64], W1['W'][64:128], Wef1, row(W1['b']),
        V1l['W'], row(V1l['b']),
        U1a['W'][:64], U1a['W'][64:96], row(U1a['b']), U1b['W'], row(U1b['b']),
        Nla['W'][:74], Nla['W'][74:138], Nla['W'][138:202], row(Nla['b']),
        Nlb['W'],
        Gla['W'][:74], Gla['W'][74:138], Gla['W'][138:202], row(Gla['b']),
        Glb['W'], row(Glb['b']),
    ]

    node_out, glob_out = pl.pallas_call(
        _kernel_body,
        out_shape=[jax.ShapeDtypeStruct((N, 64), jnp.float32),
                   jax.ShapeDtypeStruct((1, 1), jnp.float32)],
        scratch_shapes=[pltpu.VMEM((N, 128), jnp.float32),
                        pltpu.VMEM((N, 128), jnp.float32)],
    )(*args)
    return (node_out, glob_out.reshape(1))
